# baseline (speedup 1.0000x reference)
import functools

import jax
import jax.numpy as jnp
from jax import lax
from jax.experimental import pallas as pl
from jax.experimental.pallas import tpu as pltpu
from jax.experimental.pallas import tpu_sc as plsc

EPS = 1e-7
NL = 16
NEGF = -3.0e38


def _permute(x, idx):
    dnums = lax.GatherDimensionNumbers(
        offset_dims=(), collapsed_slice_dims=(0,), start_index_map=(0,))
    return lax.gather(x, idx[:, None], dnums, (1,),
                      mode=lax.GatherScatterMode.PROMISE_IN_BOUNDS)


def _sc_argmax_body(x_hbm, best_hbm, maxv_hbm, buf, best_v, maxv_v,
                    sem0, sem1, *, rows_per_worker, chunk, V, ncores,
                    row0=0):
    cid = lax.axis_index("c")
    sid = lax.axis_index("s")
    wid = sid * ncores + cid
    base = wid * rows_per_worker
    in_base = row0 + base
    nch = rows_per_worker // chunk
    vch = V // NL
    lane = lax.iota(jnp.int32, 16)
    sems = (sem0, sem1)

    def start(i, slot):
        return pltpu.async_copy(
            x_hbm.at[pl.ds(in_base + i * chunk, chunk)], buf.at[slot],
            sems[slot])

    handles = {0: start(0, 0)}

    for i in range(nch):
        slot = i % 2
        handles.pop(slot).wait()
        if i + 1 < nch:
            handles[(i + 1) % 2] = start(i + 1, (i + 1) % 2)
        bslot = buf.at[slot]

        def group_body(g, _, bslot=bslot, i=i):
            def row_body(rr, carry):
                bacc, macc = carry
                r = g * NL + rr
                row = bslot.at[r]

                def vchunk_body(cc, carry2):
                    m, ci = carry2
                    for u in range(16):
                        idx = cc * 16 + u
                        x = row[pl.ds(idx * NL, NL)]
                        gt = x > m
                        m = jnp.where(gt, x, m)
                        ci = jnp.where(gt, idx, ci)
                    return (m, ci)

                m, ci = lax.fori_loop(
                    0, vch // 16, vchunk_body,
                    (jnp.full((NL,), NEGF, jnp.float32),
                     jnp.zeros((NL,), jnp.int32)))
                mm = m
                for s in (1, 2, 4, 8):
                    mm = jnp.maximum(mm, _permute(mm, lane ^ s))
                vidx = jnp.where(m == mm, ci * NL + lane, V)
                for s in (1, 2, 4, 8):
                    vidx = jnp.minimum(vidx, _permute(vidx, lane ^ s))
                bacc = jnp.where(lane == rr, vidx, bacc)
                macc = jnp.where(lane == rr, mm, macc)
                return (bacc, macc)

            bacc, macc = lax.fori_loop(
                0, NL, row_body,
                (jnp.zeros((NL,), jnp.int32), jnp.zeros((NL,), jnp.float32)))
            off = i * chunk + g * NL
            best_v[pl.ds(off, NL)] = bacc
            maxv_v[pl.ds(off, NL)] = macc
            return 0

        lax.fori_loop(0, chunk // NL, group_body, 0)

    pltpu.sync_copy(best_v, best_hbm.at[pl.ds(base, rows_per_worker)])
    pltpu.sync_copy(maxv_v, maxv_hbm.at[pl.ds(base, rows_per_worker)])


def _make_sc_argmax(rows, V, ncores=2, row0=0, chunk=32):
    nw = 16 * ncores
    rpw = rows // nw
    mesh = plsc.VectorSubcoreMesh(core_axis_name="c", subcore_axis_name="s",
                                  num_cores=ncores)
    return pl.kernel(
        functools.partial(_sc_argmax_body, rows_per_worker=rpw, chunk=chunk,
                          V=V, ncores=ncores, row0=row0),
        out_type=[jax.ShapeDtypeStruct((rows,), jnp.int32),
                  jax.ShapeDtypeStruct((rows,), jnp.float32)],
        mesh=mesh,
        scratch_types=[pltpu.VMEM((2, chunk, V), jnp.float32),
                       pltpu.VMEM((rpw,), jnp.int32),
                       pltpu.VMEM((rpw,), jnp.float32),
                       pltpu.SemaphoreType.DMA,
                       pltpu.SemaphoreType.DMA],
    )


def _tc_main_kernel(x_ref, dec_ref, score_ref, *, T, V):
    x = x_ref[0]
    maxv = jnp.max(x, axis=1, keepdims=True)
    idx = lax.broadcasted_iota(jnp.int32, (T, V), 1)
    cand = jnp.where(x == maxv, idx, V)
    best = jnp.min(cand, axis=1, keepdims=True)
    _collapse_and_score(best, maxv, dec_ref, score_ref, T, V)


def _collapse_and_score(best, maxv, dec_ref, score_ref, T, V):
    score_ref[0] = jnp.sum(jnp.log(maxv + EPS)).reshape(1, 1)

    blank = V - 1
    prev = jnp.concatenate(
        [jnp.full((1, 1), -1, jnp.int32), best[:-1]], axis=0)
    keep = (best != prev) & (best != blank)

    c = keep.astype(jnp.float32)
    sh = 1
    while sh < T:
        c = c + jnp.concatenate(
            [jnp.zeros((sh, 1), jnp.float32), c[:-sh]], axis=0)
        sh *= 2
    pos = (c - 1.0).astype(jnp.int32)

    jidx = lax.broadcasted_iota(jnp.int32, (T, T), 1)
    onehot = ((pos == jidx) & keep).astype(jnp.float32)
    vals = (best + 1).astype(jnp.float32)
    dec_row = jnp.sum(onehot * vals, axis=0, keepdims=True) - 1.0
    dec_ref[0] = dec_row.astype(jnp.int32)


def _tc_main(inputs, k, B, T, V):
    return pl.pallas_call(
        functools.partial(_tc_main_kernel, T=T, V=V),
        grid=(B - k,),
        in_specs=[pl.BlockSpec((1, T, V), lambda b: (b + k, 0, 0))],
        out_specs=[
            pl.BlockSpec((1, 1, T), lambda b: (b, 0, 0)),
            pl.BlockSpec((1, 1, 1), lambda b: (b, 0, 0)),
        ],
        out_shape=[
            jax.ShapeDtypeStruct((B - k, 1, T), jnp.int32),
            jax.ShapeDtypeStruct((B - k, 1, 1), jnp.float32),
        ],
    )(inputs)


def _tc_post_kernel(best_ref, maxv_ref, dec_ref, score_ref, *, T, V):
    best = best_ref[0]
    maxv = maxv_ref[0]
    _collapse_and_score(best, maxv, dec_ref, score_ref, T, V)


def _tc_post(best3, maxv3, B, T, V):
    return pl.pallas_call(
        functools.partial(_tc_post_kernel, T=T, V=V),
        grid=(B,),
        in_specs=[pl.BlockSpec((1, T, 1), lambda b: (b, 0, 0)),
                  pl.BlockSpec((1, T, 1), lambda b: (b, 0, 0))],
        out_specs=[
            pl.BlockSpec((1, 1, T), lambda b: (b, 0, 0)),
            pl.BlockSpec((1, 1, 1), lambda b: (b, 0, 0)),
        ],
        out_shape=[
            jax.ShapeDtypeStruct((B, 1, T), jnp.int32),
            jax.ShapeDtypeStruct((B, 1, 1), jnp.float32),
        ],
    )(best3, maxv3)


K_SC = 16


def kernel(inputs):
    B, T, V = inputs.shape
    k = K_SC
    x2d = inputs.reshape(B * T, V)
    dec_tc, score_tc = _tc_main(inputs, k, B, T, V)
    best, maxv = _make_sc_argmax(k * T, V)(x2d)
    dec_sc, score_sc = _tc_post(best.reshape(k, T, 1), maxv.reshape(k, T, 1),
                                k, T, V)
    dec = jnp.concatenate(
        [dec_sc.reshape(k, T), dec_tc.reshape(B - k, T)], axis=0)
    score = jnp.concatenate(
        [score_sc.reshape(k, 1), score_tc.reshape(B - k, 1)], axis=0)
    return dec, score


# speedup vs baseline: 1.0389x; 1.0389x over previous
import functools

import jax
import jax.numpy as jnp
from jax import lax
from jax.experimental import pallas as pl
from jax.experimental.pallas import tpu as pltpu
from jax.experimental.pallas import tpu_sc as plsc

EPS = 1e-7
NL = 16
NEGF = -3.0e38


def _permute(x, idx):
    dnums = lax.GatherDimensionNumbers(
        offset_dims=(), collapsed_slice_dims=(0,), start_index_map=(0,))
    return lax.gather(x, idx[:, None], dnums, (1,),
                      mode=lax.GatherScatterMode.PROMISE_IN_BOUNDS)


def _sc_argmax_body(x_hbm, ci_hbm, m_hbm, buf, ci_v, m_v,
                    sem0, sem1, *, rows_per_worker, chunk, V, ncores,
                    row0=0):
    cid = lax.axis_index("c")
    sid = lax.axis_index("s")
    wid = sid * ncores + cid
    base = wid * rows_per_worker
    in_base = row0 + base
    nch = rows_per_worker // chunk
    vch = V // NL
    sems = (sem0, sem1)

    def start(i, slot):
        return pltpu.async_copy(
            x_hbm.at[pl.ds(in_base + i * chunk, chunk)], buf.at[slot],
            sems[slot])

    handles = {0: start(0, 0)}

    for i in range(nch):
        slot = i % 2
        handles.pop(slot).wait()
        if i + 1 < nch:
            handles[(i + 1) % 2] = start(i + 1, (i + 1) % 2)
        bslot = buf.at[slot]

        def row_body(rr, _, bslot=bslot, i=i):
            row = bslot.at[rr]

            def vchunk_body(cc, carry2):
                m, ci = carry2
                for u in range(16):
                    idx = cc * 16 + u
                    x = row[pl.ds(idx * NL, NL)]
                    gt = x > m
                    m = jnp.where(gt, x, m)
                    ci = jnp.where(gt, idx, ci)
                return (m, ci)

            m, ci = lax.fori_loop(
                0, vch // 16, vchunk_body,
                (jnp.full((NL,), NEGF, jnp.float32),
                 jnp.zeros((NL,), jnp.int32)))
            off16 = (i * chunk + rr) * NL
            ci_v[pl.ds(off16, NL)] = ci
            m_v[pl.ds(off16, NL)] = m
            return 0

        lax.fori_loop(0, chunk, row_body, 0)

    pltpu.sync_copy(ci_v, ci_hbm.at[pl.ds(base * NL, rows_per_worker * NL)])
    pltpu.sync_copy(m_v, m_hbm.at[pl.ds(base * NL, rows_per_worker * NL)])


def _make_sc_argmax(rows, V, ncores=2, row0=0, chunk=32):
    nw = 16 * ncores
    rpw = rows // nw
    mesh = plsc.VectorSubcoreMesh(core_axis_name="c", subcore_axis_name="s",
                                  num_cores=ncores)
    return pl.kernel(
        functools.partial(_sc_argmax_body, rows_per_worker=rpw, chunk=chunk,
                          V=V, ncores=ncores, row0=row0),
        out_type=[jax.ShapeDtypeStruct((rows * NL,), jnp.int32),
                  jax.ShapeDtypeStruct((rows * NL,), jnp.float32)],
        mesh=mesh,
        scratch_types=[pltpu.VMEM((2, chunk, V), jnp.float32),
                       pltpu.VMEM((rpw * NL,), jnp.int32),
                       pltpu.VMEM((rpw * NL,), jnp.float32),
                       pltpu.SemaphoreType.DMA,
                       pltpu.SemaphoreType.DMA],
    )


def _tc_main_kernel(x_ref, dec_ref, score_ref, *, T, V):
    x = x_ref[0]
    maxv = jnp.max(x, axis=1, keepdims=True)
    idx = lax.broadcasted_iota(jnp.int32, (T, V), 1)
    cand = jnp.where(x == maxv, idx, V)
    best = jnp.min(cand, axis=1, keepdims=True)
    _collapse_and_score(best, maxv, dec_ref, score_ref, T, V)


def _collapse_and_score(best, maxv, dec_ref, score_ref, T, V):
    score_ref[0] = jnp.sum(jnp.log(maxv + EPS)).reshape(1, 1)

    blank = V - 1
    prev = jnp.concatenate(
        [jnp.full((1, 1), -1, jnp.int32), best[:-1]], axis=0)
    keep = (best != prev) & (best != blank)

    c = keep.astype(jnp.float32)
    sh = 1
    while sh < T:
        c = c + jnp.concatenate(
            [jnp.zeros((sh, 1), jnp.float32), c[:-sh]], axis=0)
        sh *= 2
    pos = (c - 1.0).astype(jnp.int32)

    jidx = lax.broadcasted_iota(jnp.int32, (T, T), 1)
    onehot = ((pos == jidx) & keep).astype(jnp.float32)
    vals = (best + 1).astype(jnp.float32)
    dec_row = jnp.sum(onehot * vals, axis=0, keepdims=True) - 1.0
    dec_ref[0] = dec_row.astype(jnp.int32)


def _tc_main(inputs, k, B, T, V):
    return pl.pallas_call(
        functools.partial(_tc_main_kernel, T=T, V=V),
        grid=(B - k,),
        in_specs=[pl.BlockSpec((1, T, V), lambda b: (b + k, 0, 0))],
        out_specs=[
            pl.BlockSpec((1, 1, T), lambda b: (b, 0, 0)),
            pl.BlockSpec((1, 1, 1), lambda b: (b, 0, 0)),
        ],
        out_shape=[
            jax.ShapeDtypeStruct((B - k, 1, T), jnp.int32),
            jax.ShapeDtypeStruct((B - k, 1, 1), jnp.float32),
        ],
    )(inputs)


def _tc_post_kernel(ci_ref, m_ref, dec_ref, score_ref, *, T, V):
    ci = ci_ref[0]
    m = m_ref[0]
    maxv = jnp.max(m, axis=1, keepdims=True)
    lane = lax.broadcasted_iota(jnp.int32, (T, NL), 1)
    vocab = ci * NL + lane
    best = jnp.min(jnp.where(m == maxv, vocab, V), axis=1,
                   keepdims=True)
    _collapse_and_score(best, maxv, dec_ref, score_ref, T, V)


def _tc_post(ci3, m3, B, T, V):
    return pl.pallas_call(
        functools.partial(_tc_post_kernel, T=T, V=V),
        grid=(B,),
        in_specs=[pl.BlockSpec((1, T, NL), lambda b: (b, 0, 0)),
                  pl.BlockSpec((1, T, NL), lambda b: (b, 0, 0))],
        out_specs=[
            pl.BlockSpec((1, 1, T), lambda b: (b, 0, 0)),
            pl.BlockSpec((1, 1, 1), lambda b: (b, 0, 0)),
        ],
        out_shape=[
            jax.ShapeDtypeStruct((B, 1, T), jnp.int32),
            jax.ShapeDtypeStruct((B, 1, 1), jnp.float32),
        ],
    )(ci3, m3)


K_SC = 16


def kernel(inputs):
    B, T, V = inputs.shape
    k = K_SC
    x2d = inputs.reshape(B * T, V)
    dec_tc, score_tc = _tc_main(inputs, k, B, T, V)
    ci, m = _make_sc_argmax(k * T, V)(x2d)
    dec_sc, score_sc = _tc_post(ci.reshape(k, T, NL), m.reshape(k, T, NL),
                                k, T, V)
    dec = jnp.concatenate(
        [dec_sc.reshape(k, T), dec_tc.reshape(B - k, T)], axis=0)
    score = jnp.concatenate(
        [score_sc.reshape(k, 1), score_tc.reshape(B - k, 1)], axis=0)
    return dec, score
